# baseline (speedup 1.0000x reference)
import functools

import jax
import jax.numpy as jnp
from jax import lax
from jax.experimental import pallas as pl
from jax.experimental.pallas import tpu as pltpu
from jax.experimental.pallas import tpu_sc as plsc

B, VIS, D, TOTAL = 512, 64, 96, 256
ROWS_OUT = TOTAL + 1
IMG_ROWS = VIS + 1
MASK_ROW = B * IMG_ROWS
NW = 32
SPT = B // NW
HALF = TOTAL // 2
L = 16


def _sc_gather(table, idx):
    mesh = plsc.VectorSubcoreMesh(core_axis_name="c", subcore_axis_name="s")

    @functools.partial(
        pl.kernel,
        out_type=jax.ShapeDtypeStruct((B * ROWS_OUT, D), jnp.float32),
        mesh=mesh,
        compiler_params=pltpu.CompilerParams(use_tc_tiling_on_sc=False),
        scratch_types=[
            pltpu.VMEM((TOTAL,), jnp.int32),
            pltpu.VMEM((2, HALF), jnp.int32),
            pltpu.VMEM((TOTAL, D), jnp.float32),
            pltpu.VMEM((L, D), jnp.float32),
            pltpu.SemaphoreType.DMA,
        ],
    )
    def k(table_hbm, idx_hbm, out_hbm, idx_v, flat_v, rows_v, gt_v, sem):
        wid = lax.axis_index("s") * 2 + lax.axis_index("c")
        base = wid * SPT

        lanes = lax.iota(jnp.int32, L)
        gt_src = (base + lanes) * IMG_ROWS
        gt_dst = (base + lanes) * ROWS_OUT
        pltpu.async_copy(table_hbm.at[gt_src], gt_v, sem).wait()
        pltpu.async_copy(gt_v, out_hbm.at[gt_dst], sem).wait()

        def body(s, carry):
            b = base + s
            pltpu.sync_copy(idx_hbm.at[b], idx_v)
            off = b * IMG_ROWS + 1
            for c in range(TOTAL // L):
                j = idx_v[pl.ds(c * L, L)]
                flat = jnp.where(j < VIS, off + j, MASK_ROW)
                flat_v[c // (HALF // L), pl.ds((c % (HALF // L)) * L, L)] = flat
            for h in range(2):
                pltpu.async_copy(
                    table_hbm.at[flat_v.at[h]],
                    rows_v.at[pl.ds(h * HALF, HALF)],
                    sem,
                ).wait()
            pltpu.sync_copy(rows_v, out_hbm.at[pl.ds(b * ROWS_OUT + 1, TOTAL)])
            return carry

        lax.fori_loop(0, SPT, body, 0)

    return k(table, idx)


def kernel(img, img_revert_idx, mask_token):
    table = jnp.concatenate([img.reshape(B * IMG_ROWS, D), mask_token], axis=0)
    out = _sc_gather(table, img_revert_idx)
    return out.reshape(B, ROWS_OUT, D)


# speedup vs baseline: 1.0028x; 1.0028x over previous
import functools

import jax
import jax.numpy as jnp
from jax import lax
from jax.experimental import pallas as pl
from jax.experimental.pallas import tpu as pltpu
from jax.experimental.pallas import tpu_sc as plsc

B, VIS, D, TOTAL = 512, 64, 96, 256
ROWS_OUT = TOTAL + 1
IMG_ROWS = VIS + 1
MASK_ROW = B * IMG_ROWS
NW = 32
SPT = B // NW
HALF = TOTAL // 2
L = 16
NBUF = 4


def _sc_gather(table, idx):
    mesh = plsc.VectorSubcoreMesh(core_axis_name="c", subcore_axis_name="s")

    @functools.partial(
        pl.kernel,
        out_type=jax.ShapeDtypeStruct((B * ROWS_OUT, D), jnp.float32),
        mesh=mesh,
        compiler_params=pltpu.CompilerParams(use_tc_tiling_on_sc=False),
        scratch_types=[
            pltpu.VMEM((SPT, TOTAL), jnp.int32),
            pltpu.VMEM((NBUF, 2, HALF), jnp.int32),
            pltpu.VMEM((NBUF, TOTAL, D), jnp.float32),
            pltpu.VMEM((L, D), jnp.float32),
            pltpu.SemaphoreType.DMA((NBUF,)),
            pltpu.SemaphoreType.DMA((NBUF,)),
            pltpu.SemaphoreType.DMA,
        ],
    )
    def k(table_hbm, idx_hbm, out_hbm, idx_v, flat_v, rows_v, gt_v,
          gsem, ssem, isem):
        wid = lax.axis_index("s") * 2 + lax.axis_index("c")
        base = wid * SPT
        lanes = lax.iota(jnp.int32, L)
        bvec = base + lanes

        idx_cp = pltpu.async_copy(idx_hbm.at[bvec], idx_v, isem)

        gt_cp = pltpu.async_copy(table_hbm.at[bvec * IMG_ROWS], gt_v, isem)
        idx_cp.wait()
        gt_cp.wait()
        pltpu.async_copy(gt_v, out_hbm.at[bvec * ROWS_OUT], isem).wait()

        def compute_flat(s):
            slot = s % NBUF
            off = (base + s) * IMG_ROWS + 1
            for c in range(TOTAL // L):
                j = idx_v[s, pl.ds(c * L, L)]
                flat = jnp.where(j < VIS, off + j, MASK_ROW)
                flat_v[slot, c // (HALF // L), pl.ds((c % (HALF // L)) * L, L)] = flat

        def fire_gather(s):
            slot = s % NBUF
            return [
                pltpu.async_copy(
                    table_hbm.at[flat_v.at[slot, h]],
                    rows_v.at[slot, pl.ds(h * HALF, HALF)],
                    gsem.at[slot],
                )
                for h in range(2)
            ]

        gathers = {}
        scatters = {}
        for s in range(NBUF):
            compute_flat(s)
            gathers[s] = fire_gather(s)

        for s in range(SPT):
            slot = s % NBUF
            for cp in gathers[s]:
                cp.wait()
            scatters[s] = pltpu.async_copy(
                rows_v.at[slot],
                out_hbm.at[pl.ds((base + s) * ROWS_OUT + 1, TOTAL)],
                ssem.at[slot],
            )
            if s + NBUF < SPT:
                compute_flat(s + NBUF)
                scatters[s].wait()
                gathers[s + NBUF] = fire_gather(s + NBUF)

        for s in range(SPT - NBUF, SPT):
            scatters[s].wait()

    return k(table, idx)


def kernel(img, img_revert_idx, mask_token):
    table = jnp.concatenate([img.reshape(B * IMG_ROWS, D), mask_token], axis=0)
    out = _sc_gather(table, img_revert_idx)
    return out.reshape(B, ROWS_OUT, D)


# speedup vs baseline: 10.5600x; 10.5309x over previous
import functools

import jax
import jax.numpy as jnp
from jax import lax
from jax.experimental import pallas as pl
from jax.experimental.pallas import tpu as pltpu
from jax.experimental.pallas import tpu_sc as plsc

B, VIS, D, TOTAL = 512, 64, 96, 256
ROWS_OUT = TOTAL + 1
IMG_ROWS = VIS + 1
NW = 32
SPT = B // NW
L = 16
DC = D // L
STG = IMG_ROWS + 1
MASK_LOCAL = IMG_ROWS


def _sc_revert(img_flat, idx, mt):
    mesh = plsc.VectorSubcoreMesh(core_axis_name="c", subcore_axis_name="s")

    @functools.partial(
        pl.kernel,
        out_type=jax.ShapeDtypeStruct((B * ROWS_OUT, D), jnp.float32),
        mesh=mesh,
        compiler_params=pltpu.CompilerParams(use_tc_tiling_on_sc=False),
        scratch_types=[
            pltpu.VMEM((SPT, TOTAL), jnp.int32),
            pltpu.VMEM((2 * STG, D), jnp.float32),
            pltpu.VMEM((2 * ROWS_OUT, D), jnp.float32),
            pltpu.SemaphoreType.DMA((2,)),
            pltpu.SemaphoreType.DMA((2,)),
            pltpu.SemaphoreType.DMA,
        ],
    )
    def k(img_hbm, idx_hbm, mt_hbm, out_hbm, idx_v, stage, outb,
          gsem, ssem, isem):
        wid = lax.axis_index("s") * 2 + lax.axis_index("c")
        base = wid * SPT

        idx_cp = pltpu.async_copy(
            idx_hbm.at[pl.ds(base, SPT)], idx_v, isem)
        mt0 = pltpu.async_copy(mt_hbm, stage.at[pl.ds(MASK_LOCAL, 1)], isem)
        mt1 = pltpu.async_copy(
            mt_hbm, stage.at[pl.ds(STG + MASK_LOCAL, 1)], isem)

        def stage_in(s):
            return pltpu.async_copy(
                img_hbm.at[pl.ds((base + s) * IMG_ROWS, IMG_ROWS)],
                stage.at[pl.ds((s % 2) * STG, IMG_ROWS)],
                gsem.at[s % 2],
            )

        stg_cp = {0: stage_in(0)}
        idx_cp.wait()
        mt0.wait()
        mt1.wait()

        out_cp = {}
        for s in range(SPT):
            sb = (s % 2) * STG
            ob = (s % 2) * ROWS_OUT
            if s + 1 < SPT:
                stg_cp[s + 1] = stage_in(s + 1)
            stg_cp[s].wait()
            if s >= 2:
                out_cp[s - 2].wait()

            for c in range(DC):
                outb[ob, pl.ds(c * L, L)] = stage[sb, pl.ds(c * L, L)]

            def group(g, carry):
                j = idx_v[s, pl.ds(g * L, L)]
                srcs = jnp.where(j < VIS, sb + 1 + j, sb + MASK_LOCAL)
                trow = ob + 1 + g * L
                for kk in range(L):
                    src = srcs[kk]
                    for c in range(DC):
                        outb[trow + kk, pl.ds(c * L, L)] = (
                            stage[src, pl.ds(c * L, L)])
                return carry

            lax.fori_loop(0, TOTAL // L, group, 0)

            out_cp[s] = pltpu.async_copy(
                outb.at[pl.ds(ob, ROWS_OUT)],
                out_hbm.at[pl.ds((base + s) * ROWS_OUT, ROWS_OUT)],
                ssem.at[s % 2],
            )

        out_cp[SPT - 2].wait()
        out_cp[SPT - 1].wait()

    return k(img_flat, idx, mt)


def kernel(img, img_revert_idx, mask_token):
    out = _sc_revert(img.reshape(B * IMG_ROWS, D), img_revert_idx, mask_token)
    return out.reshape(B, ROWS_OUT, D)


# speedup vs baseline: 16.7864x; 1.5896x over previous
import functools

import jax
import jax.numpy as jnp
from jax import lax
from jax.experimental import pallas as pl
from jax.experimental.pallas import tpu as pltpu
from jax.experimental.pallas import tpu_sc as plsc

B, VIS, D, TOTAL = 512, 64, 96, 256
ROWS_OUT = TOTAL + 1
IMG_ROWS = VIS + 1
NW = 32
SPT = B // NW
L = 16
DC = D // L
MASK_LOCAL = 72
STG = MASK_LOCAL + 8
OUTB = 264


def _sc_revert(img, idx, mt):
    mesh = plsc.VectorSubcoreMesh(core_axis_name="c", subcore_axis_name="s")

    @functools.partial(
        pl.kernel,
        out_type=jax.ShapeDtypeStruct((B, ROWS_OUT, D), jnp.float32),
        mesh=mesh,
        scratch_types=[
            pltpu.VMEM((SPT, TOTAL), jnp.int32),
            pltpu.VMEM((2 * STG, D), jnp.float32),
            pltpu.VMEM((2 * OUTB, D), jnp.float32),
            pltpu.SemaphoreType.DMA((2,)),
            pltpu.SemaphoreType.DMA((2,)),
            pltpu.SemaphoreType.DMA,
        ],
    )
    def k(img_hbm, idx_hbm, mt_hbm, out_hbm, idx_v, stage, outb,
          gsem, ssem, isem):
        wid = lax.axis_index("s") * 2 + lax.axis_index("c")
        base = wid * SPT

        idx_cp = pltpu.async_copy(
            idx_hbm.at[pl.ds(base, SPT)], idx_v, isem)
        mt0 = pltpu.async_copy(mt_hbm, stage.at[pl.ds(MASK_LOCAL, 1)], isem)
        mt1 = pltpu.async_copy(
            mt_hbm, stage.at[pl.ds(STG + MASK_LOCAL, 1)], isem)

        def stage_in(s):
            return pltpu.async_copy(
                img_hbm.at[base + s],
                stage.at[pl.ds((s % 2) * STG, IMG_ROWS)],
                gsem.at[s % 2],
            )

        stg_cp = {0: stage_in(0)}
        idx_cp.wait()
        mt0.wait()
        mt1.wait()

        out_cp = {}
        for s in range(SPT):
            sb = (s % 2) * STG
            ob = (s % 2) * OUTB
            if s + 1 < SPT:
                stg_cp[s + 1] = stage_in(s + 1)
            stg_cp[s].wait()
            if s >= 2:
                out_cp[s - 2].wait()

            for c in range(DC):
                outb[ob, pl.ds(c * L, L)] = stage[sb, pl.ds(c * L, L)]

            def group(g, carry):
                j = idx_v[s, pl.ds(g * L, L)]
                srcs = jnp.where(j < VIS, sb + 1 + j, sb + MASK_LOCAL)
                trow = ob + 1 + g * L
                for kk in range(L):
                    src = srcs[kk]
                    for c in range(DC):
                        outb[trow + kk, pl.ds(c * L, L)] = (
                            stage[src, pl.ds(c * L, L)])
                return carry

            lax.fori_loop(0, TOTAL // L, group, 0)

            out_cp[s] = pltpu.async_copy(
                outb.at[pl.ds(ob, ROWS_OUT)],
                out_hbm.at[base + s],
                ssem.at[s % 2],
            )

        out_cp[SPT - 2].wait()
        out_cp[SPT - 1].wait()

    return k(img, idx, mt)


def kernel(img, img_revert_idx, mask_token):
    return _sc_revert(img, img_revert_idx, mask_token)


# speedup vs baseline: 24.9694x; 1.4875x over previous
import functools

import jax
import jax.numpy as jnp
from jax import lax
from jax.experimental import pallas as pl
from jax.experimental.pallas import tpu as pltpu
from jax.experimental.pallas import tpu_sc as plsc

B, VIS, D, TOTAL = 512, 64, 96, 256
ROWS_OUT = TOTAL + 1
IMG_ROWS = VIS + 1
NW = 32
SPT = B // NW
L = 16
DC = D // L
MASK_LOCAL = 72
STG = MASK_LOCAL + 8
OUTB = 264


def _sc_revert(img, idx, mt):
    mesh = plsc.VectorSubcoreMesh(core_axis_name="c", subcore_axis_name="s")

    @functools.partial(
        pl.kernel,
        out_type=jax.ShapeDtypeStruct((B, ROWS_OUT, D), jnp.float32),
        mesh=mesh,
        scratch_types=[
            pltpu.VMEM((SPT, TOTAL), jnp.int32),
            pltpu.VMEM((2 * STG, D), jnp.float32),
            pltpu.VMEM((2 * OUTB, D), jnp.float32),
            pltpu.SemaphoreType.DMA((2,)),
            pltpu.SemaphoreType.DMA((2,)),
            pltpu.SemaphoreType.DMA,
        ],
    )
    def k(img_hbm, idx_hbm, mt_hbm, out_hbm, idx_v, stage, outb,
          gsem, ssem, isem):
        wid = lax.axis_index("s") * 2 + lax.axis_index("c")
        base = wid * SPT

        idx_cp = pltpu.async_copy(
            idx_hbm.at[pl.ds(base, SPT)], idx_v, isem)
        mt0 = pltpu.async_copy(mt_hbm, stage.at[pl.ds(MASK_LOCAL, 1)], isem)
        mt1 = pltpu.async_copy(
            mt_hbm, stage.at[pl.ds(STG + MASK_LOCAL, 1)], isem)

        def stage_cp(s, slot):
            return pltpu.make_async_copy(
                img_hbm.at[base + s],
                stage.at[pl.ds(slot * STG, IMG_ROWS)],
                gsem.at[slot],
            )

        def out_cp(s, slot):
            return pltpu.make_async_copy(
                outb.at[pl.ds(slot * OUTB, ROWS_OUT)],
                out_hbm.at[base + s],
                ssem.at[slot],
            )

        stage_cp(0, 0).start()
        idx_cp.wait()
        mt0.wait()
        mt1.wait()

        def body(s, carry):
            slot = lax.rem(s, 2)
            sb = slot * STG
            ob = slot * OUTB

            stage_cp(s, slot).wait()

            @pl.when(s + 1 < SPT)
            def _():
                stage_cp(s + 1, 1 - slot).start()

            @pl.when(s >= 2)
            def _():
                out_cp(s - 2, slot).wait()

            for c in range(DC):
                outb[ob, pl.ds(c * L, L)] = stage[sb, pl.ds(c * L, L)]

            @plsc.parallel_loop(0, TOTAL // L, unroll=2)
            def group(g):
                j = idx_v[s, pl.ds(g * L, L)]
                srcs = jnp.where(j < VIS, sb + 1 + j, sb + MASK_LOCAL)
                trow = ob + 1 + g * L
                for kk in range(L):
                    src = srcs[kk]
                    for c in range(DC):
                        outb[trow + kk, pl.ds(c * L, L)] = (
                            stage[src, pl.ds(c * L, L)])

            out_cp(s, slot).start()
            return carry

        lax.fori_loop(0, SPT, body, 0)

        out_cp(SPT - 2, 0).wait()
        out_cp(SPT - 1, 1).wait()

    return k(img, idx, mt)


def kernel(img, img_revert_idx, mask_token):
    return _sc_revert(img, img_revert_idx, mask_token)


# speedup vs baseline: 26.9237x; 1.0783x over previous
import functools

import jax
import jax.numpy as jnp
from jax import lax
from jax.experimental import pallas as pl
from jax.experimental.pallas import tpu as pltpu
from jax.experimental.pallas import tpu_sc as plsc

B, VIS, D, TOTAL = 512, 64, 96, 256
ROWS_OUT = TOTAL + 1
IMG_ROWS = VIS + 1
L = 16
DGRP = 8
BCH = 128
MASK_ROW = IMG_ROWS
NITEMS = 3
CHUNK = 16
NCH = 8


def _sc_revert(img_t, idx, mt):
    mesh = plsc.VectorSubcoreMesh(core_axis_name="c", subcore_axis_name="s")

    @functools.partial(
        pl.kernel,
        out_type=jax.ShapeDtypeStruct((ROWS_OUT, D, B), jnp.float32),
        mesh=mesh,
        compiler_params=pltpu.CompilerParams(needs_layout_passes=False),
        scratch_types=[
            pltpu.VMEM((IMG_ROWS + 1, DGRP, BCH), jnp.float32),
            pltpu.VMEM((BCH, BCH), jnp.int32),
            pltpu.VMEM((2, CHUNK, DGRP, BCH), jnp.float32),
            pltpu.VMEM((1, D), jnp.float32),
            pltpu.SemaphoreType.DMA,
            pltpu.SemaphoreType.DMA((2,)),
            pltpu.SemaphoreType.DMA,
        ],
    )
    def k(img_hbm, idx_hbm, mt_hbm, out_hbm, stage, idxv, outb, mtv,
          gsem, ssem, msem):
        wid = lax.axis_index("s") * 2 + lax.axis_index("c")
        lanes = lax.iota(jnp.int32, L)

        pltpu.async_copy(mt_hbm, mtv, msem).wait()

        for i in range(NITEMS):
            item = wid * NITEMS + i
            dg = item // 8
            rem = item - dg * 8
            d0 = pl.multiple_of(dg * DGRP, DGRP)
            b0 = pl.multiple_of((rem // 2) * BCH, BCH)
            th = rem - (rem // 2) * 2
            to0 = th * (NCH * CHUNK)

            stage_cp = pltpu.async_copy(
                img_hbm.at[:, pl.ds(d0, DGRP), pl.ds(b0, BCH)],
                stage.at[pl.ds(0, IMG_ROWS)],
                gsem,
            )
            idx_cp = pltpu.async_copy(
                idx_hbm.at[pl.ds(b0, BCH), pl.ds(to0, BCH)],
                idxv,
                msem,
            )

            for dloc in range(DGRP):
                md = plsc.load_gather(
                    mtv, [jnp.zeros((L,), jnp.int32),
                          jnp.full((L,), d0 + dloc, jnp.int32)])
                for g in range(BCH // L):
                    stage[MASK_ROW, dloc, pl.ds(g * L, L)] = md

            stage_cp.wait()
            idx_cp.wait()

            @pl.when(th == 0)
            def _():
                pltpu.async_copy(
                    stage.at[pl.ds(0, 1)],
                    out_hbm.at[pl.ds(0, 1), pl.ds(d0, DGRP), pl.ds(b0, BCH)],
                    msem,
                ).wait()

            def drain_cp(c, slot):
                return pltpu.make_async_copy(
                    outb.at[slot],
                    out_hbm.at[pl.ds(to0 + c * CHUNK + 1, CHUNK),
                               pl.ds(d0, DGRP), pl.ds(b0, BCH)],
                    ssem.at[slot],
                )

            def chunk_body(c, carry):
                slot = lax.rem(c, 2)

                @pl.when(c >= 2)
                def _():
                    drain_cp(c - 2, slot).wait()

                @plsc.parallel_loop(0, CHUNK, unroll=2)
                def row(r):
                    tl = c * CHUNK + r
                    for g in range(BCH // L):
                        bl = g * L + lanes
                        j = plsc.load_gather(
                            idxv, [bl, jnp.full((L,), tl, jnp.int32)])
                        srcs = jnp.where(j < VIS, j + 1, MASK_ROW)
                        for dloc in range(DGRP):
                            v = plsc.load_gather(
                                stage,
                                [srcs, jnp.full((L,), dloc, jnp.int32), bl])
                            outb[slot, r, dloc, pl.ds(g * L, L)] = v

                drain_cp(c, slot).start()
                return carry

            lax.fori_loop(0, NCH, chunk_body, 0)
            drain_cp(NCH - 2, 0).wait()
            drain_cp(NCH - 1, 1).wait()

    return k(img_t, idx, mt)


def kernel(img, img_revert_idx, mask_token):
    img_t = jnp.transpose(img, (1, 2, 0))
    out_t = _sc_revert(img_t, img_revert_idx, mask_token)
    return jnp.transpose(out_t, (2, 0, 1))


# speedup vs baseline: 36.0049x; 1.3373x over previous
import functools

import jax
import jax.numpy as jnp
from jax import lax
from jax.experimental import pallas as pl
from jax.experimental.pallas import tpu as pltpu
from jax.experimental.pallas import tpu_sc as plsc

B, VIS, D, TOTAL = 512, 64, 96, 256
ROWS_OUT = TOTAL + 1
IMG_ROWS = VIS + 1
L = 16
DGRP = 8
BCH = 128
MASK_ROW = IMG_ROWS
NITEMS = 3
CHUNK = 16
NCH = 8


def _sc_revert(img_t, idx, mt):
    mesh = plsc.VectorSubcoreMesh(core_axis_name="c", subcore_axis_name="s")

    @functools.partial(
        pl.kernel,
        out_type=jax.ShapeDtypeStruct((ROWS_OUT, D, B), jnp.float32),
        mesh=mesh,
        compiler_params=pltpu.CompilerParams(needs_layout_passes=False),
        scratch_types=[
            pltpu.VMEM((IMG_ROWS + 1, DGRP, BCH), jnp.float32),
            pltpu.VMEM((BCH, BCH), jnp.int32),
            pltpu.VMEM((2, CHUNK, DGRP, BCH), jnp.float32),
            pltpu.VMEM((1, D), jnp.float32),
            pltpu.SemaphoreType.DMA,
            pltpu.SemaphoreType.DMA((2,)),
            pltpu.SemaphoreType.DMA,
        ],
    )
    def k(img_hbm, idx_hbm, mt_hbm, out_hbm, stage, idxv, outb, mtv,
          gsem, ssem, msem):
        wid = lax.axis_index("s") * 2 + lax.axis_index("c")
        lanes = lax.iota(jnp.int32, L)

        pltpu.async_copy(mt_hbm, mtv, msem).wait()

        def item_body(i, carry):
            item = wid * NITEMS + i
            dg = item // 8
            rem = item - dg * 8
            d0 = pl.multiple_of(dg * DGRP, DGRP)
            b0 = pl.multiple_of((rem // 2) * BCH, BCH)
            th = rem - (rem // 2) * 2
            to0 = th * (NCH * CHUNK)

            stage_cp = pltpu.async_copy(
                img_hbm.at[:, pl.ds(d0, DGRP), pl.ds(b0, BCH)],
                stage.at[pl.ds(0, IMG_ROWS)],
                gsem,
            )
            idx_cp = pltpu.async_copy(
                idx_hbm.at[pl.ds(b0, BCH), pl.ds(to0, BCH)],
                idxv,
                msem,
            )

            for dloc in range(DGRP):
                md = plsc.load_gather(
                    mtv, [jnp.zeros((L,), jnp.int32),
                          jnp.full((L,), d0 + dloc, jnp.int32)])
                for g in range(BCH // L):
                    stage[MASK_ROW, dloc, pl.ds(g * L, L)] = md

            stage_cp.wait()
            idx_cp.wait()

            @pl.when(th == 0)
            def _():
                pltpu.async_copy(
                    stage.at[pl.ds(0, 1)],
                    out_hbm.at[pl.ds(0, 1), pl.ds(d0, DGRP), pl.ds(b0, BCH)],
                    msem,
                ).wait()

            def drain_cp(c, slot):
                return pltpu.make_async_copy(
                    outb.at[slot],
                    out_hbm.at[pl.ds(to0 + c * CHUNK + 1, CHUNK),
                               pl.ds(d0, DGRP), pl.ds(b0, BCH)],
                    ssem.at[slot],
                )

            def chunk_body(c, carry):
                slot = lax.rem(c, 2)

                @pl.when(c >= 2)
                def _():
                    drain_cp(c - 2, slot).wait()

                @plsc.parallel_loop(0, CHUNK, unroll=4)
                def row(r):
                    tl = c * CHUNK + r
                    for g in range(BCH // L):
                        bl = g * L + lanes
                        j = plsc.load_gather(
                            idxv, [bl, jnp.full((L,), tl, jnp.int32)])
                        srcs = jnp.where(j < VIS, j + 1, MASK_ROW)
                        for dloc in range(DGRP):
                            v = plsc.load_gather(
                                stage,
                                [srcs, jnp.full((L,), dloc, jnp.int32), bl])
                            outb[slot, r, dloc, pl.ds(g * L, L)] = v

                drain_cp(c, slot).start()
                return carry

            lax.fori_loop(0, NCH, chunk_body, 0)
            drain_cp(NCH - 2, 0).wait()
            drain_cp(NCH - 1, 1).wait()
            return carry

        lax.fori_loop(0, NITEMS, item_body, 0)

    return k(img_t, idx, mt)


def kernel(img, img_revert_idx, mask_token):
    img_t = jnp.transpose(img, (1, 2, 0))
    out_t = _sc_revert(img_t, img_revert_idx, mask_token)
    return jnp.transpose(out_t, (2, 0, 1))
